# baseline (device time: 72065 ns/iter reference)
import jax
import jax.numpy as jnp
import numpy as np
from jax import lax
from jax.experimental import pallas as pl
from jax.experimental.pallas import tpu as pltpu

N_DEV = 32
CW_HOPS = 16
CCW_HOPS = 15
SUBS = 8


def _ring_tables():
    yz = [(0, 0), (0, 1), (0, 2), (0, 3), (1, 3), (1, 2), (1, 1), (2, 1),
          (2, 2), (2, 3), (3, 3), (3, 2), (3, 1), (3, 0), (2, 0), (1, 0)]
    plane = {(0, 0): 0, (1, 0): 1, (1, 1): 2, (0, 1): 3,
             (0, 2): 4, (1, 2): 5, (1, 3): 6, (0, 3): 7}
    coords = []
    for i, (y, z) in enumerate(yz):
        for x in ((0, 1) if i % 2 == 0 else (1, 0)):
            coords.append((x, y, z))
    assert all(
        sum(abs(u - v) for u, v in zip(coords[r], coords[(r + 1) % 32])) == 1
        for r in range(32)
    )
    perm = [z * 8 + plane[(x, y)] for (x, y, z) in coords]
    inv = [0] * N_DEV
    for r, m in enumerate(perm):
        inv[m] = r
    return np.array(perm, np.int32), np.array(inv, np.int32)


_PERM, _INV = _ring_tables()


def kernel(x, w_mat, scale_x, scale_w):
    m_global, k_sh = x.shape
    _, n = w_mat.shape
    m_blk = m_global // N_DEV
    sub = n // SUBS
    s2 = (scale_x * scale_w).reshape(1, 1)

    def body(x_ref, w_ref, s_ref, perm_ref, inv_ref, out_ref,
             xg_ref, wg_cw, wg_ccw,
             x_send_sems, x_recv_sems,
             cw_send_sems, cw_recv_sems, ccw_send_sems, ccw_recv_sems):
        my = lax.axis_index("i")
        rp = inv_ref[my]
        right = perm_ref[lax.rem(rp + 1, N_DEV)]
        left = perm_ref[lax.rem(rp + N_DEV - 1, N_DEV)]

        barrier_sem = pltpu.get_barrier_semaphore()
        for o in range(1, N_DEV):
            pl.semaphore_signal(
                barrier_sem, inc=1,
                device_id=(lax.rem(my + o, N_DEV),),
                device_id_type=pl.DeviceIdType.MESH,
            )
        pl.semaphore_wait(barrier_sem, N_DEV - 1)

        def x_rdma(o):
            d = lax.rem(my + o, N_DEV)
            return pltpu.make_async_remote_copy(
                src_ref=x_ref.at[pl.ds(d * m_blk, m_blk), :],
                dst_ref=xg_ref.at[my],
                send_sem=x_send_sems.at[o - 1],
                recv_sem=x_recv_sems.at[my],
                device_id=(d,),
                device_id_type=pl.DeviceIdType.MESH,
            )

        def x_recv(src):
            return pltpu.make_async_remote_copy(
                src_ref=x_ref.at[pl.ds(0, m_blk), :],
                dst_ref=xg_ref.at[src],
                send_sem=x_send_sems.at[0],
                recv_sem=x_recv_sems.at[src],
                device_id=(src,),
                device_id_type=pl.DeviceIdType.MESH,
            )

        def w_rdma(t, h, cw):
            buf, ssem, rsem, tgt = (
                (wg_cw, cw_send_sems, cw_recv_sems, right) if cw
                else (wg_ccw, ccw_send_sems, ccw_recv_sems, left)
            )
            src = (
                w_ref.at[:, pl.ds(t * sub, sub)] if h == 0
                else buf.at[t, h - 1]
            )
            return pltpu.make_async_remote_copy(
                src_ref=src,
                dst_ref=buf.at[t, h],
                send_sem=ssem.at[t, h],
                recv_sem=rsem.at[t, h],
                device_id=(tgt,),
                device_id_type=pl.DeviceIdType.MESH,
            )

        for t in range(SUBS):
            w_rdma(t, 0, True).start()
            w_rdma(t, 0, False).start()
        for o in range(1, N_DEV):
            x_rdma(o).start()
        for o in range(1, N_DEV):
            x_recv(lax.rem(my + o, N_DEV)).wait_recv()

        xb_my = x_ref[pl.ds(my * m_blk, m_blk), :]
        acc = [
            lax.dot_general(
                xb_my, w_ref[:, t * sub:(t + 1) * sub],
                (((1,), (0,)), ((), ())),
                preferred_element_type=jnp.int32,
            )
            for t in range(SUBS)
        ]

        def gemm_in(origin, buf, t, h, a):
            xb = xg_ref[origin]
            return a + lax.dot_general(
                xb, buf[t, h], (((1,), (0,)), ((), ())),
                preferred_element_type=jnp.int32,
            )

        for h in range(CW_HOPS):
            for t in range(SUBS):
                w_rdma(t, h, True).wait_recv()
                if h + 1 < CW_HOPS:
                    w_rdma(t, h + 1, True).start()
                if h < CCW_HOPS:
                    w_rdma(t, h, False).wait_recv()
                    if h + 1 < CCW_HOPS:
                        w_rdma(t, h + 1, False).start()
            o_cw = perm_ref[lax.rem(rp + N_DEV - 1 - h, N_DEV)]
            for t in range(SUBS):
                acc[t] = gemm_in(o_cw, wg_cw, t, h, acc[t])
            if h < CCW_HOPS:
                o_ccw = perm_ref[lax.rem(rp + 1 + h, N_DEV)]
                for t in range(SUBS):
                    acc[t] = gemm_in(o_ccw, wg_ccw, t, h, acc[t])

        scale = s_ref[0, 0]
        for t in range(SUBS):
            out_ref[:, t * sub:(t + 1) * sub] = jnp.maximum(
                acc[t].astype(jnp.float32) * scale, 0.0
            )

        for o in range(1, N_DEV):
            x_rdma(o).wait_send()
        for t in range(SUBS):
            for h in range(CW_HOPS):
                w_rdma(t, h, True).wait_send()
            for h in range(CCW_HOPS):
                w_rdma(t, h, False).wait_send()

    return pl.pallas_call(
        body,
        out_shape=jax.ShapeDtypeStruct((m_blk, n), jnp.float32),
        in_specs=[
            pl.BlockSpec(memory_space=pltpu.VMEM),
            pl.BlockSpec(memory_space=pltpu.VMEM),
            pl.BlockSpec(memory_space=pltpu.SMEM),
            pl.BlockSpec(memory_space=pltpu.SMEM),
            pl.BlockSpec(memory_space=pltpu.SMEM),
        ],
        out_specs=pl.BlockSpec(memory_space=pltpu.VMEM),
        scratch_shapes=[
            pltpu.VMEM((N_DEV, m_blk, k_sh), jnp.int8),
            pltpu.VMEM((SUBS, CW_HOPS, k_sh, sub), jnp.int8),
            pltpu.VMEM((SUBS, CCW_HOPS, k_sh, sub), jnp.int8),
            pltpu.SemaphoreType.DMA((N_DEV - 1,)),
            pltpu.SemaphoreType.DMA((N_DEV,)),
            pltpu.SemaphoreType.DMA((SUBS, CW_HOPS)),
            pltpu.SemaphoreType.DMA((SUBS, CW_HOPS)),
            pltpu.SemaphoreType.DMA((SUBS, CCW_HOPS)),
            pltpu.SemaphoreType.DMA((SUBS, CCW_HOPS)),
        ],
        compiler_params=pltpu.CompilerParams(
            collective_id=0, vmem_limit_bytes=64 * 1024 * 1024
        ),
    )(x, w_mat, s2, jnp.asarray(_PERM), jnp.asarray(_INV))


# device time: 69440 ns/iter; 1.0378x vs baseline; 1.0378x over previous
import jax
import jax.numpy as jnp
import numpy as np
from jax import lax
from jax.experimental import pallas as pl
from jax.experimental.pallas import tpu as pltpu

N_DEV = 32
CW_HOPS = 16
CCW_HOPS = 15
SUBS = 8


def _ring_tables():
    yz = [(0, 0), (0, 1), (0, 2), (0, 3), (1, 3), (1, 2), (1, 1), (2, 1),
          (2, 2), (2, 3), (3, 3), (3, 2), (3, 1), (3, 0), (2, 0), (1, 0)]
    plane = {(0, 0): 0, (1, 0): 1, (1, 1): 2, (0, 1): 3,
             (0, 2): 4, (1, 2): 5, (1, 3): 6, (0, 3): 7}
    coords = []
    for i, (y, z) in enumerate(yz):
        for x in ((0, 1) if i % 2 == 0 else (1, 0)):
            coords.append((x, y, z))
    assert all(
        sum(abs(u - v) for u, v in zip(coords[r], coords[(r + 1) % 32])) == 1
        for r in range(32)
    )
    perm = [z * 8 + plane[(x, y)] for (x, y, z) in coords]
    inv = [0] * N_DEV
    for r, m in enumerate(perm):
        inv[m] = r
    return np.array(perm, np.int32), np.array(inv, np.int32)


_PERM, _INV = _ring_tables()


def kernel(x, w_mat, scale_x, scale_w):
    m_global, k_sh = x.shape
    _, n = w_mat.shape
    m_blk = m_global // N_DEV
    sub = n // SUBS
    s2 = (scale_x * scale_w).reshape(1, 1)

    def body(x_ref, w_ref, s_ref, perm_ref, inv_ref, out_ref,
             xg_ref, wg_cw, wg_ccw,
             x_send_sems, x_recv_sems,
             cw_send_sems, cw_recv_sems, ccw_send_sems, ccw_recv_sems):
        my = lax.axis_index("i")
        rp = inv_ref[my]
        right = perm_ref[lax.rem(rp + 1, N_DEV)]
        left = perm_ref[lax.rem(rp + N_DEV - 1, N_DEV)]

        barrier_sem = pltpu.get_barrier_semaphore()
        for o in range(1, N_DEV):
            pl.semaphore_signal(
                barrier_sem, inc=1,
                device_id=(lax.rem(my + o, N_DEV),),
                device_id_type=pl.DeviceIdType.MESH,
            )
        pl.semaphore_wait(barrier_sem, N_DEV - 1)

        def x_rdma(dist, cw):
            rpt = lax.rem(rp + (dist if cw else N_DEV - dist), N_DEV)
            tgt = perm_ref[rpt]
            slot = (dist - 1) if cw else (CW_HOPS + dist - 1)
            return pltpu.make_async_remote_copy(
                src_ref=x_ref.at[pl.ds(tgt * m_blk, m_blk), :],
                dst_ref=xg_ref.at[my],
                send_sem=x_send_sems.at[slot],
                recv_sem=x_recv_sems.at[my],
                device_id=(tgt,),
                device_id_type=pl.DeviceIdType.MESH,
            )

        def x_recv(src):
            return pltpu.make_async_remote_copy(
                src_ref=x_ref.at[pl.ds(0, m_blk), :],
                dst_ref=xg_ref.at[src],
                send_sem=x_send_sems.at[0],
                recv_sem=x_recv_sems.at[src],
                device_id=(src,),
                device_id_type=pl.DeviceIdType.MESH,
            )

        def w_rdma(t, h, cw):
            buf, ssem, rsem, tgt = (
                (wg_cw, cw_send_sems, cw_recv_sems, right) if cw
                else (wg_ccw, ccw_send_sems, ccw_recv_sems, left)
            )
            src = (
                w_ref.at[:, pl.ds(t * sub, sub)] if h == 0
                else buf.at[t, h - 1]
            )
            return pltpu.make_async_remote_copy(
                src_ref=src,
                dst_ref=buf.at[t, h],
                send_sem=ssem.at[t, h],
                recv_sem=rsem.at[t, h],
                device_id=(tgt,),
                device_id_type=pl.DeviceIdType.MESH,
            )

        for t in range(SUBS):
            w_rdma(t, 0, True).start()
            w_rdma(t, 0, False).start()
        for d in range(1, 5):
            x_rdma(d, True).start()
            x_rdma(d, False).start()

        xb_my = x_ref[pl.ds(my * m_blk, m_blk), :]
        acc = [
            lax.dot_general(
                xb_my, w_ref[:, t * sub:(t + 1) * sub],
                (((1,), (0,)), ((), ())),
                preferred_element_type=jnp.int32,
            )
            for t in range(SUBS)
        ]

        def gemm_in(origin, buf, t, h, a):
            xb = xg_ref[origin]
            return a + lax.dot_general(
                xb, buf[t, h], (((1,), (0,)), ((), ())),
                preferred_element_type=jnp.int32,
            )

        for h in range(CW_HOPS):
            for t in range(SUBS):
                w_rdma(t, h, True).wait_recv()
                if h + 1 < CW_HOPS:
                    w_rdma(t, h + 1, True).start()
                if h < CCW_HOPS:
                    w_rdma(t, h, False).wait_recv()
                    if h + 1 < CCW_HOPS:
                        w_rdma(t, h + 1, False).start()
            d = h + 5
            if d <= CW_HOPS:
                x_rdma(d, True).start()
            if d <= CCW_HOPS:
                x_rdma(d, False).start()
            o_cw = perm_ref[lax.rem(rp + N_DEV - 1 - h, N_DEV)]
            x_recv(o_cw).wait_recv()
            for t in range(SUBS):
                acc[t] = gemm_in(o_cw, wg_cw, t, h, acc[t])
            if h < CCW_HOPS:
                o_ccw = perm_ref[lax.rem(rp + 1 + h, N_DEV)]
                x_recv(o_ccw).wait_recv()
                for t in range(SUBS):
                    acc[t] = gemm_in(o_ccw, wg_ccw, t, h, acc[t])

        scale = s_ref[0, 0]
        for t in range(SUBS):
            out_ref[:, t * sub:(t + 1) * sub] = jnp.maximum(
                acc[t].astype(jnp.float32) * scale, 0.0
            )

        for d in range(1, CW_HOPS + 1):
            x_rdma(d, True).wait_send()
        for d in range(1, CCW_HOPS + 1):
            x_rdma(d, False).wait_send()
        for t in range(SUBS):
            for h in range(CW_HOPS):
                w_rdma(t, h, True).wait_send()
            for h in range(CCW_HOPS):
                w_rdma(t, h, False).wait_send()

    return pl.pallas_call(
        body,
        out_shape=jax.ShapeDtypeStruct((m_blk, n), jnp.float32),
        in_specs=[
            pl.BlockSpec(memory_space=pltpu.VMEM),
            pl.BlockSpec(memory_space=pltpu.VMEM),
            pl.BlockSpec(memory_space=pltpu.SMEM),
            pl.BlockSpec(memory_space=pltpu.SMEM),
            pl.BlockSpec(memory_space=pltpu.SMEM),
        ],
        out_specs=pl.BlockSpec(memory_space=pltpu.VMEM),
        scratch_shapes=[
            pltpu.VMEM((N_DEV, m_blk, k_sh), jnp.int8),
            pltpu.VMEM((SUBS, CW_HOPS, k_sh, sub), jnp.int8),
            pltpu.VMEM((SUBS, CCW_HOPS, k_sh, sub), jnp.int8),
            pltpu.SemaphoreType.DMA((N_DEV - 1,)),
            pltpu.SemaphoreType.DMA((N_DEV,)),
            pltpu.SemaphoreType.DMA((SUBS, CW_HOPS)),
            pltpu.SemaphoreType.DMA((SUBS, CW_HOPS)),
            pltpu.SemaphoreType.DMA((SUBS, CCW_HOPS)),
            pltpu.SemaphoreType.DMA((SUBS, CCW_HOPS)),
        ],
        compiler_params=pltpu.CompilerParams(
            collective_id=0, vmem_limit_bytes=64 * 1024 * 1024
        ),
    )(x, w_mat, s2, jnp.asarray(_PERM), jnp.asarray(_INV))
